# Initial kernel scaffold; baseline (speedup 1.0000x reference)
import jax
import jax.numpy as jnp
from jax.experimental import pallas as pl


def kernel(x, batch):
    raise NotImplementedError("write your pallas kernel here")


# speedup vs baseline: 2.9259x; 2.9259x over previous
import functools

import jax
import jax.numpy as jnp
from jax import lax
from jax.experimental import pallas as pl
from jax.experimental.pallas import tpu as pltpu
from jax.experimental.pallas import tpu_sc as plsc

NSEG = 128
NROWS = 50000
D = 512
C = 80
NBLK = NROWS // C
NC = 2
NS = 16
NW = NC * NS
BASE = NBLK // NW
EXTRA = NBLK - BASE * NW
LANES = 16


def _sc_segment_sums(x, batch_i32):
  mesh = plsc.VectorSubcoreMesh(core_axis_name="c", subcore_axis_name="s")

  @functools.partial(
      pl.kernel,
      mesh=mesh,
      compiler_params=pltpu.CompilerParams(needs_layout_passes=False),
      out_type=[
          jax.ShapeDtypeStruct((NW, NSEG * D), jnp.float32),
          jax.ShapeDtypeStruct((NW, NSEG * LANES), jnp.float32),
      ],
      scratch_types=[
          pltpu.VMEM((C,), jnp.int32),
          pltpu.VMEM((C, D), jnp.float32),
          pltpu.VMEM((NSEG * D,), jnp.float32),
          pltpu.VMEM((NSEG * LANES,), jnp.float32),
      ],
  )
  def seg_sum(x_hbm, b_hbm, sums_hbm, cnts_hbm, idx_v, rows_v, acc_v, cacc_v):
    cid = lax.axis_index("c")
    sid = lax.axis_index("s")
    wid = sid * NC + cid

    zeros = jnp.zeros((LANES,), jnp.float32)
    ones = jnp.ones((LANES,), jnp.float32)
    lane_iota = lax.iota(jnp.int32, LANES)

    def zbody(i, carry):
      for j in range(D // LANES):
        acc_v[pl.ds(i * D + j * LANES, LANES)] = zeros
      cacc_v[pl.ds(i * LANES, LANES)] = zeros
      return carry

    lax.fori_loop(0, NSEG, zbody, 0)

    nblk_w = jnp.where(wid < EXTRA, BASE + 1, BASE)
    start = wid * BASE + jnp.minimum(wid, EXTRA)

    def body(i, carry):
      blk = start + i
      pltpu.sync_copy(b_hbm.at[pl.ds(blk * C, C)], idx_v)
      pltpu.sync_copy(x_hbm.at[pl.ds(blk * C, C)], rows_v)

      def row_body(g, carry2):
        ids16 = idx_v[pl.ds(g * LANES, LANES)]
        plsc.addupdate_scatter(cacc_v, [ids16 * LANES + lane_iota], ones)
        for l in range(LANES):
          base = lax.broadcast(ids16[l] * D, (LANES,)) + lane_iota
          r = g * LANES + l
          for j in range(D // LANES):
            vals = rows_v[r, pl.ds(j * LANES, LANES)]
            plsc.addupdate_scatter(acc_v, [base + (j * LANES)], vals)
        return carry2

      lax.fori_loop(0, C // LANES, row_body, 0)
      return carry

    lax.fori_loop(0, nblk_w, body, 0)

    pltpu.sync_copy(acc_v, sums_hbm.at[wid])
    pltpu.sync_copy(cacc_v, cnts_hbm.at[wid])

  return seg_sum(x, batch_i32)


def _combine(sums, cnts):
  def body(s_ref, c_ref, o_ref):
    s = jnp.sum(s_ref[...], axis=0)
    c = jnp.sum(c_ref[...], axis=(0, 2))
    o_ref[...] = s / jnp.maximum(c, 1.0)[:, None]

  return pl.pallas_call(
      body,
      out_shape=jax.ShapeDtypeStruct((NSEG, D), jnp.float32),
  )(sums, cnts)


@jax.jit
def kernel(x, batch):
  sums, cnts = _sc_segment_sums(x, batch.astype(jnp.int32))
  sums = sums.reshape(NW, NSEG, D)
  cnts = cnts.reshape(NW, NSEG, LANES)
  return _combine(sums, cnts)


# speedup vs baseline: 4.8450x; 1.6559x over previous
import functools

import jax
import jax.numpy as jnp
from jax import lax
from jax.experimental import pallas as pl
from jax.experimental.pallas import tpu as pltpu
from jax.experimental.pallas import tpu_sc as plsc

NSEG = 128
NROWS = 50000
D = 512
LANES = 16
C = 48
G = C // LANES
NC = 2
NS = 16
NW = NC * NS
Q = (-(-NROWS // NW) + 7) // 8 * 8
NWIN = -(-Q // C)


def _tree_sum(vs):
  while len(vs) > 1:
    vs = [a + b for a, b in zip(vs[::2], vs[1::2])]
  return vs[0]


def _sc_segment_sums(x, batch_i32):
  mesh = plsc.VectorSubcoreMesh(core_axis_name="c", subcore_axis_name="s")

  @functools.partial(
      pl.kernel,
      mesh=mesh,
      compiler_params=pltpu.CompilerParams(needs_layout_passes=False),
      out_type=[
          jax.ShapeDtypeStruct((NW, NSEG * D), jnp.float32),
          jax.ShapeDtypeStruct((NW, NSEG * LANES), jnp.float32),
      ],
      scratch_types=[
          pltpu.VMEM((C,), jnp.int32),
          pltpu.VMEM((C,), jnp.int32),
          pltpu.VMEM((C, D), jnp.float32),
          pltpu.VMEM((C, D), jnp.float32),
          pltpu.VMEM((NSEG * D,), jnp.float32),
          pltpu.VMEM((NSEG * LANES,), jnp.float32),
          pltpu.SemaphoreType.DMA,
          pltpu.SemaphoreType.DMA,
          pltpu.SemaphoreType.DMA,
          pltpu.SemaphoreType.DMA,
      ],
  )
  def seg_sum(x_hbm, b_hbm, sums_hbm, cnts_hbm,
              idx0, idx1, rows0, rows1, acc_v, cacc_v,
              semi0, semi1, semx0, semx1):
    cid = lax.axis_index("c")
    sid = lax.axis_index("s")
    wid = sid * NC + cid

    zeros = jnp.zeros((LANES,), jnp.float32)
    ones = jnp.ones((LANES,), jnp.float32)
    lane_iota = lax.iota(jnp.int32, LANES)

    def zbody(i, carry):
      for j in range(D // LANES):
        acc_v[pl.ds(i * D + j * LANES, LANES)] = zeros
      cacc_v[pl.ds(i * LANES, LANES)] = zeros
      return carry

    lax.fori_loop(0, NSEG, zbody, 0)

    start = wid * Q
    end = jnp.minimum(start + Q, NROWS)

    idx_b = [idx0, idx1]
    rows_b = [rows0, rows1]
    semi = [semi0, semi1]
    semx = [semx0, semx1]

    def wstart(i):
      return jnp.minimum(start + i * C, NROWS - C)

    def issue(i, b):
      ws = wstart(i)
      pltpu.async_copy(b_hbm.at[pl.ds(ws, C)], idx_b[b], semi[b])
      pltpu.async_copy(x_hbm.at[pl.ds(ws, C)], rows_b[b], semx[b])

    def wait(i, b):
      ws = wstart(i)
      pltpu.make_async_copy(b_hbm.at[pl.ds(ws, C)], idx_b[b], semi[b]).wait()
      pltpu.make_async_copy(x_hbm.at[pl.ds(ws, C)], rows_b[b], semx[b]).wait()

    def process(i, b):
      lo = start + i * C
      ws = wstart(i)
      ib = idx_b[b]
      rb = rows_b[b]

      def gbody(g, carry):
        r0 = ws + g * LANES
        ids16 = ib[pl.ds(g * LANES, LANES)]
        gr = lax.broadcast(r0, (LANES,)) + lane_iota
        vmask = (gr >= lo) & (gr < end)
        plsc.addupdate_scatter(
            cacc_v, [ids16 * LANES + lane_iota], ones, mask=vmask)

        full = (ids16[0] == ids16[LANES - 1]) & (r0 >= lo) & (r0 + LANES <= end)

        @pl.when(full)
        def _fast():
          addr = lax.broadcast(ids16[0] * D, (LANES,)) + lane_iota
          for j in range(D // LANES):
            s = _tree_sum(
                [rb[g * LANES + l, pl.ds(j * LANES, LANES)]
                 for l in range(LANES)])
            plsc.addupdate_scatter(acc_v, [addr + (j * LANES)], s)

        @pl.when(jnp.logical_not(full))
        def _slow():
          idsD = ids16 * D
          for l in range(LANES):
            rl = r0 + l
            inb = (rl >= lo) & (rl < end)
            m = lax.broadcast(inb, (LANES,))
            seg = lax.broadcast(idsD[l], (LANES,)) + lane_iota
            for j in range(D // LANES):
              plsc.addupdate_scatter(
                  acc_v, [seg + (j * LANES)],
                  rb[g * LANES + l, pl.ds(j * LANES, LANES)], mask=m)

        return carry

      @pl.when(lo < end)
      def _():
        lax.fori_loop(0, G, gbody, 0)

    issue(0, 0)

    def pbody(p, carry):
      w = p * 2
      issue(w + 1, 1)
      wait(w, 0)
      process(w, 0)
      issue(w + 2, 0)
      wait(w + 1, 1)
      process(w + 1, 1)
      return carry

    lax.fori_loop(0, (NWIN - 1) // 2, pbody, 0)
    wait(NWIN - 1, 0)
    process(NWIN - 1, 0)

    pltpu.sync_copy(acc_v, sums_hbm.at[wid])
    pltpu.sync_copy(cacc_v, cnts_hbm.at[wid])

  return seg_sum(x, batch_i32)


def _combine(sums, cnts):
  def body(s_ref, c_ref, o_ref):
    s = jnp.sum(s_ref[...], axis=0)
    c = jnp.sum(c_ref[...], axis=(0, 2))
    o_ref[...] = s / jnp.maximum(c, 1.0)[:, None]

  return pl.pallas_call(
      body,
      out_shape=jax.ShapeDtypeStruct((NSEG, D), jnp.float32),
  )(sums, cnts)


@jax.jit
def kernel(x, batch):
  sums, cnts = _sc_segment_sums(x, batch.astype(jnp.int32))
  sums = sums.reshape(NW, NSEG, D)
  cnts = cnts.reshape(NW, NSEG, LANES)
  return _combine(sums, cnts)


# speedup vs baseline: 7.6621x; 1.5815x over previous
import functools

import jax
import jax.numpy as jnp
from jax import lax
from jax.experimental import pallas as pl
from jax.experimental.pallas import tpu as pltpu
from jax.experimental.pallas import tpu_sc as plsc

NSEG = 128
NROWS = 50000
D = 512
LANES = 16
C = 48
G = C // LANES
NC = 2
NS = 16
NW = NC * NS
Q = (-(-NROWS // NW) + 7) // 8 * 8
NWIN = -(-Q // C)


def _tree_sum(vs):
  while len(vs) > 1:
    vs = [a + b for a, b in zip(vs[::2], vs[1::2])]
  return vs[0]


def _sc_segment_sums(x, batch_i32):
  mesh = plsc.VectorSubcoreMesh(core_axis_name="c", subcore_axis_name="s")

  @functools.partial(
      pl.kernel,
      mesh=mesh,
      compiler_params=pltpu.CompilerParams(needs_layout_passes=False),
      out_type=[
          jax.ShapeDtypeStruct((NW, NSEG * D), jnp.float32),
          jax.ShapeDtypeStruct((NW, NSEG * LANES), jnp.float32),
      ],
      scratch_types=[
          pltpu.VMEM((C,), jnp.int32),
          pltpu.VMEM((C,), jnp.int32),
          pltpu.VMEM((C, D), jnp.float32),
          pltpu.VMEM((C, D), jnp.float32),
          pltpu.VMEM((NSEG * D,), jnp.float32),
          pltpu.VMEM((NSEG * LANES,), jnp.float32),
          pltpu.SemaphoreType.DMA,
          pltpu.SemaphoreType.DMA,
          pltpu.SemaphoreType.DMA,
          pltpu.SemaphoreType.DMA,
      ],
  )
  def seg_sum(x_hbm, b_hbm, sums_hbm, cnts_hbm,
              idx0, idx1, rows0, rows1, acc_v, cacc_v,
              semi0, semi1, semx0, semx1):
    cid = lax.axis_index("c")
    sid = lax.axis_index("s")
    wid = sid * NC + cid

    zeros = jnp.zeros((LANES,), jnp.float32)
    ones = jnp.ones((LANES,), jnp.float32)
    lane_iota = lax.iota(jnp.int32, LANES)

    def zbody(i, carry):
      for j in range(D // LANES):
        acc_v[pl.ds(i * D + j * LANES, LANES)] = zeros
      cacc_v[pl.ds(i * LANES, LANES)] = zeros
      return carry

    lax.fori_loop(0, NSEG, zbody, 0)

    start = wid * Q
    end = jnp.minimum(start + Q, NROWS)

    idx_b = [idx0, idx1]
    rows_b = [rows0, rows1]
    semi = [semi0, semi1]
    semx = [semx0, semx1]

    def wstart(i):
      return jnp.minimum(start + i * C, NROWS - C)

    def issue(i, b):
      ws = wstart(i)
      pltpu.async_copy(b_hbm.at[pl.ds(ws, C)], idx_b[b], semi[b])
      pltpu.async_copy(x_hbm.at[pl.ds(ws, C)], rows_b[b], semx[b])

    def wait(i, b):
      ws = wstart(i)
      pltpu.make_async_copy(b_hbm.at[pl.ds(ws, C)], idx_b[b], semi[b]).wait()
      pltpu.make_async_copy(x_hbm.at[pl.ds(ws, C)], rows_b[b], semx[b]).wait()

    def process(i, b):
      lo = start + i * C
      ws = wstart(i)
      ib = idx_b[b]
      rb = rows_b[b]

      ids_first = ib[pl.ds(0, LANES)]
      ids_last = ib[pl.ds(C - LANES, LANES)]
      wuni = ((ids_first[0] == ids_last[LANES - 1])
              & (ws >= lo) & (ws + C <= end))

      @pl.when(wuni)
      def _window_uniform():
        for g in range(G):
          ids16 = ib[pl.ds(g * LANES, LANES)]
          plsc.addupdate_scatter(cacc_v, [ids16 * LANES + lane_iota], ones)
        addr = lax.broadcast(ids_first[0] * D, (LANES,)) + lane_iota

        def jbody(j, carry):
          parts = []
          for g in range(G):
            parts.append(_tree_sum(
                [rb[g * LANES + l, pl.ds(j * LANES, LANES)]
                 for l in range(LANES)]))
          plsc.addupdate_scatter(acc_v, [addr + j * LANES], _tree_sum(parts))
          return carry

        lax.fori_loop(0, D // LANES, jbody, 0)

      def gbody(g, carry):
        r0 = ws + g * LANES
        ids16 = ib[pl.ds(g * LANES, LANES)]
        gr = lax.broadcast(r0, (LANES,)) + lane_iota
        vmask = (gr >= lo) & (gr < end)
        plsc.addupdate_scatter(
            cacc_v, [ids16 * LANES + lane_iota], ones, mask=vmask)

        full = (ids16[0] == ids16[LANES - 1]) & (r0 >= lo) & (r0 + LANES <= end)

        @pl.when(full)
        def _fast():
          addr = lax.broadcast(ids16[0] * D, (LANES,)) + lane_iota
          for j in range(D // LANES):
            s = _tree_sum(
                [rb[g * LANES + l, pl.ds(j * LANES, LANES)]
                 for l in range(LANES)])
            plsc.addupdate_scatter(acc_v, [addr + (j * LANES)], s)

        @pl.when(jnp.logical_not(full))
        def _slow():
          idsD = ids16 * D

          def lbody(l, carry3):
            rl = r0 + l
            inb = (rl >= lo) & (rl < end)
            m = lax.broadcast(inb, (LANES,))
            seg = idsD.at[lax.broadcast(l, (LANES,))].get(
                mode="promise_in_bounds") + lane_iota
            for j in range(D // LANES):
              plsc.addupdate_scatter(
                  acc_v, [seg + (j * LANES)],
                  rb[g * LANES + l, pl.ds(j * LANES, LANES)], mask=m)
            return carry3

          lax.fori_loop(0, LANES, lbody, 0)

        return carry

      @pl.when(jnp.logical_not(wuni) & (lo < end))
      def _():
        lax.fori_loop(0, G, gbody, 0)

    issue(0, 0)

    def pbody(p, carry):
      w = p * 2
      issue(w + 1, 1)
      wait(w, 0)
      process(w, 0)
      issue(w + 2, 0)
      wait(w + 1, 1)
      process(w + 1, 1)
      return carry

    lax.fori_loop(0, (NWIN - 1) // 2, pbody, 0)
    wait(NWIN - 1, 0)
    process(NWIN - 1, 0)

    pltpu.sync_copy(acc_v, sums_hbm.at[wid])
    pltpu.sync_copy(cacc_v, cnts_hbm.at[wid])

  return seg_sum(x, batch_i32)


def _combine(sums, cnts):
  def body(s_ref, c_ref, o_ref):
    s = jnp.sum(s_ref[...], axis=0)
    c = jnp.sum(c_ref[...], axis=(0, 2))
    o_ref[...] = s / jnp.maximum(c, 1.0)[:, None]

  return pl.pallas_call(
      body,
      out_shape=jax.ShapeDtypeStruct((NSEG, D), jnp.float32),
  )(sums, cnts)


@jax.jit
def kernel(x, batch):
  sums, cnts = _sc_segment_sums(x, batch.astype(jnp.int32))
  sums = sums.reshape(NW, NSEG, D)
  cnts = cnts.reshape(NW, NSEG, LANES)
  return _combine(sums, cnts)
